# baseline (speedup 1.0000x reference)
import functools

import numpy as np
import jax
import jax.numpy as jnp
from jax import lax
from jax.experimental import pallas as pl
from jax.experimental.pallas import tpu as pltpu
from jax.experimental.pallas import tpu_sc as plsc

_B = 16384
_T = 65536
_BLK = 2048
_ITERS = _BLK // 16
_NBLK = _B // _BLK
_NC, _NS = 2, 16

_P = [np.int32(1),
      np.int32(2654435761 - (1 << 32)),
      np.int32(805459861),
      np.int32(3674653429 - (1 << 32))]
_RES = np.asarray([np.floor(16.0 * 1.5 ** l) for l in range(16)], np.float32)


def _sc_encode(zt, ptab, res_arr):
    mesh = plsc.VectorSubcoreMesh(core_axis_name="c", subcore_axis_name="s")

    @functools.partial(
        pl.kernel,
        mesh=mesh,
        out_type=jax.ShapeDtypeStruct((128, _B), jnp.float32),
        compiler_params=pltpu.CompilerParams(needs_layout_passes=False),
        scratch_types=[
            pltpu.VMEM((_T,), jnp.int32),
            pltpu.VMEM((4, _BLK), jnp.float32),
            pltpu.VMEM((2, _BLK), jnp.float32),
            pltpu.VMEM((16,), jnp.float32),
        ],
    )
    def k(zt_h, ptab_h, res_h, out_h, tab_v, z_v, o_v, res_v):
        wid = lax.axis_index("s") * _NC + lax.axis_index("c")
        pltpu.sync_copy(res_h, res_v)
        for kk in range(2):
            combo = wid * 2 + kk
            g = combo // 16
            l = combo % 16
            pltpu.sync_copy(ptab_h.at[g, l], tab_v)
            res = plsc.load_gather(res_v, [jnp.full((16,), l, jnp.int32)])

            def blk_body(b, _):
                for d in range(4):
                    pltpu.sync_copy(zt_h.at[g * 4 + d, pl.ds(b * _BLK, _BLK)],
                                    z_v.at[d])

                @plsc.parallel_loop(0, _BLK, 16, unroll=4)
                def it_body(s):
                    zz = [z_v[d, pl.ds(s, 16)] for d in range(4)]
                    sig = [1.0 / (1.0 + jnp.exp(-t)) for t in zz]
                    pos = [sg * res for sg in sig]
                    pii = [p.astype(jnp.int32) for p in pos]
                    fr = [p - q.astype(jnp.float32) for p, q in zip(pos, pii)]
                    gr = [1.0 - f for f in fr]
                    sa = [pii[d] * _P[d] for d in range(4)]
                    sb = [sa[d] + _P[d] for d in range(4)]
                    M = np.int32(0xFFFF)
                    h01 = [(sa[0] ^ sa[1]) & M, (sb[0] ^ sa[1]) & M,
                           (sa[0] ^ sb[1]) & M, (sb[0] ^ sb[1]) & M]
                    h23 = [(sa[2] ^ sa[3]) & M, (sb[2] ^ sa[3]) & M,
                           (sa[2] ^ sb[3]) & M, (sb[2] ^ sb[3]) & M]
                    w01 = [gr[0] * gr[1], fr[0] * gr[1],
                           gr[0] * fr[1], fr[0] * fr[1]]
                    w23 = [gr[2] * gr[3], fr[2] * gr[3],
                           gr[2] * fr[3], fr[2] * fr[3]]
                    acc0 = jnp.zeros((16,), jnp.float32)
                    acc1 = jnp.zeros((16,), jnp.float32)
                    for c in range(16):
                        a, bb = c & 3, c >> 2
                        idx = h01[a] ^ h23[bb]
                        pk = plsc.load_gather(tab_v, [idx])
                        v0 = lax.bitcast_convert_type(pk << 16, jnp.float32)
                        v1 = lax.bitcast_convert_type(pk & np.int32(-65536),
                                                      jnp.float32)
                        w = w01[a] * w23[bb]
                        acc0 = acc0 + w * v0
                        acc1 = acc1 + w * v1
                    o_v[0, pl.ds(s, 16)] = acc0
                    o_v[1, pl.ds(s, 16)] = acc1

                r = combo * 2
                pltpu.sync_copy(o_v.at[0], out_h.at[r, pl.ds(b * _BLK, _BLK)])
                pltpu.sync_copy(o_v.at[1],
                                out_h.at[r + 1, pl.ds(b * _BLK, _BLK)])
                return 0

            lax.fori_loop(0, _NBLK, blk_body, 0)

    return k(zt, ptab, res_arr)


def _tc_mlp(enc_t, W1, b1, W2, b2):
    BB = 2048

    def body(e_ref, w1_ref, b1_ref, w2_ref, b2_ref, o_ref):
        e = e_ref[...]
        h = lax.dot_general(e, w1_ref[...], (((0,), (0,)), ((), ())),
                            preferred_element_type=jnp.float32)
        h = h + b1_ref[...]
        h = jnp.where(h >= 0, h, 0.01 * h)
        o = lax.dot_general(h, w2_ref[...], (((1,), (0,)), ((), ())),
                            preferred_element_type=jnp.float32)
        o_ref[...] = o + b2_ref[...]

    return pl.pallas_call(
        body,
        grid=(_B // BB,),
        in_specs=[
            pl.BlockSpec((128, BB), lambda i: (0, i)),
            pl.BlockSpec((128, 256), lambda i: (0, 0)),
            pl.BlockSpec((1, 256), lambda i: (0, 0)),
            pl.BlockSpec((256, 64), lambda i: (0, 0)),
            pl.BlockSpec((1, 64), lambda i: (0, 0)),
        ],
        out_specs=pl.BlockSpec((BB, 64), lambda i: (i, 0)),
        out_shape=jax.ShapeDtypeStruct((_B, 64), jnp.float32),
    )(enc_t, W1, b1.reshape(1, -1), W2, b2.reshape(1, -1))


def kernel(z, tables, W1, b1, W2, b2):
    zt = z.T
    t32 = lax.bitcast_convert_type(tables, jnp.uint32)
    lo, hi = t32[..., 0], t32[..., 1]
    lo_r = (lo + 0x7FFF + ((lo >> 16) & 1)) >> 16
    hi_r = (hi + 0x7FFF + ((hi >> 16) & 1)) & jnp.uint32(0xFFFF0000)
    packed = lax.bitcast_convert_type(hi_r | lo_r, jnp.int32)
    enc_t = _sc_encode(zt, packed, jnp.asarray(_RES))
    return _tc_mlp(enc_t, W1, b1, W2, b2)


# speedup vs baseline: 1.0355x; 1.0355x over previous
import functools

import numpy as np
import jax
import jax.numpy as jnp
from jax import lax
from jax.experimental import pallas as pl
from jax.experimental.pallas import tpu as pltpu
from jax.experimental.pallas import tpu_sc as plsc

_B = 16384
_T = 65536
_BLK = 2048
_ITERS = _BLK // 16
_NBLK = _B // _BLK
_NC, _NS = 2, 16

_P = [np.int32(1),
      np.int32(2654435761 - (1 << 32)),
      np.int32(805459861),
      np.int32(3674653429 - (1 << 32))]
_RES = np.asarray([np.floor(16.0 * 1.5 ** l) for l in range(16)], np.float32)


def _sc_encode(zt, ptab, res_arr):
    mesh = plsc.VectorSubcoreMesh(core_axis_name="c", subcore_axis_name="s")

    @functools.partial(
        pl.kernel,
        mesh=mesh,
        out_type=jax.ShapeDtypeStruct((128, _B), jnp.float32),
        compiler_params=pltpu.CompilerParams(needs_layout_passes=False),
        scratch_types=[
            pltpu.VMEM((_T,), jnp.int32),
            pltpu.VMEM((4, _BLK), jnp.float32),
            pltpu.VMEM((2, _BLK), jnp.float32),
            pltpu.VMEM((16,), jnp.float32),
        ],
    )
    def k(zt_h, ptab_h, res_h, out_h, tab_v, z_v, o_v, res_v):
        wid = lax.axis_index("s") * _NC + lax.axis_index("c")
        pltpu.sync_copy(res_h, res_v)
        for kk in range(2):
            combo = wid * 2 + kk
            g = combo // 16
            l = combo % 16
            pltpu.sync_copy(ptab_h.at[g, l], tab_v)
            res = plsc.load_gather(res_v, [jnp.full((16,), l, jnp.int32)])

            def blk_body(b, _):
                for d in range(4):
                    pltpu.sync_copy(zt_h.at[g * 4 + d, pl.ds(b * _BLK, _BLK)],
                                    z_v.at[d])

                @plsc.parallel_loop(0, _BLK, 16, unroll=2)
                def it_body(s):
                    zz = [z_v[d, pl.ds(s, 16)] for d in range(4)]
                    sig = [1.0 / (1.0 + jnp.exp(-t)) for t in zz]
                    pos = [sg * res for sg in sig]
                    pii = [p.astype(jnp.int32) for p in pos]
                    fr = [p - q.astype(jnp.float32) for p, q in zip(pos, pii)]
                    gr = [1.0 - f for f in fr]
                    sa = [pii[d] * _P[d] for d in range(4)]
                    sb = [sa[d] + _P[d] for d in range(4)]
                    M = np.int32(0xFFFF)
                    h01 = [(sa[0] ^ sa[1]) & M, (sb[0] ^ sa[1]) & M,
                           (sa[0] ^ sb[1]) & M, (sb[0] ^ sb[1]) & M]
                    h23 = [(sa[2] ^ sa[3]) & M, (sb[2] ^ sa[3]) & M,
                           (sa[2] ^ sb[3]) & M, (sb[2] ^ sb[3]) & M]
                    w01 = [gr[0] * gr[1], fr[0] * gr[1],
                           gr[0] * fr[1], fr[0] * fr[1]]
                    w23 = [gr[2] * gr[3], fr[2] * gr[3],
                           gr[2] * fr[3], fr[2] * fr[3]]
                    acc0 = jnp.zeros((16,), jnp.float32)
                    acc1 = jnp.zeros((16,), jnp.float32)
                    for c in range(16):
                        a, bb = c & 3, c >> 2
                        idx = h01[a] ^ h23[bb]
                        pk = plsc.load_gather(tab_v, [idx])
                        v0 = lax.bitcast_convert_type(pk << 16, jnp.float32)
                        v1 = lax.bitcast_convert_type(pk & np.int32(-65536),
                                                      jnp.float32)
                        w = w01[a] * w23[bb]
                        acc0 = acc0 + w * v0
                        acc1 = acc1 + w * v1
                    o_v[0, pl.ds(s, 16)] = acc0
                    o_v[1, pl.ds(s, 16)] = acc1

                r = combo * 2
                pltpu.sync_copy(o_v.at[0], out_h.at[r, pl.ds(b * _BLK, _BLK)])
                pltpu.sync_copy(o_v.at[1],
                                out_h.at[r + 1, pl.ds(b * _BLK, _BLK)])
                return 0

            lax.fori_loop(0, _NBLK, blk_body, 0)

    return k(zt, ptab, res_arr)


def _tc_mlp(enc_t, W1, b1, W2, b2):
    BB = 2048

    def body(e_ref, w1_ref, b1_ref, w2_ref, b2_ref, o_ref):
        e = e_ref[...]
        h = lax.dot_general(e, w1_ref[...], (((0,), (0,)), ((), ())),
                            preferred_element_type=jnp.float32)
        h = h + b1_ref[...]
        h = jnp.where(h >= 0, h, 0.01 * h)
        o = lax.dot_general(h, w2_ref[...], (((1,), (0,)), ((), ())),
                            preferred_element_type=jnp.float32)
        o_ref[...] = o + b2_ref[...]

    return pl.pallas_call(
        body,
        grid=(_B // BB,),
        in_specs=[
            pl.BlockSpec((128, BB), lambda i: (0, i)),
            pl.BlockSpec((128, 256), lambda i: (0, 0)),
            pl.BlockSpec((1, 256), lambda i: (0, 0)),
            pl.BlockSpec((256, 64), lambda i: (0, 0)),
            pl.BlockSpec((1, 64), lambda i: (0, 0)),
        ],
        out_specs=pl.BlockSpec((BB, 64), lambda i: (i, 0)),
        out_shape=jax.ShapeDtypeStruct((_B, 64), jnp.float32),
    )(enc_t, W1, b1.reshape(1, -1), W2, b2.reshape(1, -1))


def kernel(z, tables, W1, b1, W2, b2):
    zt = z.T
    t32 = lax.bitcast_convert_type(tables, jnp.uint32)
    lo, hi = t32[..., 0], t32[..., 1]
    lo_r = (lo + 0x7FFF + ((lo >> 16) & 1)) >> 16
    hi_r = (hi + 0x7FFF + ((hi >> 16) & 1)) & jnp.uint32(0xFFFF0000)
    packed = lax.bitcast_convert_type(hi_r | lo_r, jnp.int32)
    enc_t = _sc_encode(zt, packed, jnp.asarray(_RES))
    return _tc_mlp(enc_t, W1, b1, W2, b2)
